# baseline (speedup 1.0000x reference)
import jax
import jax.numpy as jnp
from jax.experimental import pallas as pl

_FILL = 1e-15
_N = 32


def _max_layer_kernel(x_ref, o_ref):
    x = x_ref[0]
    R, L = x.shape
    G = L // _N
    TOP = _N * _N // L

    lane = jax.lax.broadcasted_iota(jnp.int32, (R, L), 1)
    sub = lane & (_N - 1)

    v = x
    for s in (16, 8, 4, 2, 1):
        v = jnp.maximum(v, jnp.roll(v, -s, axis=1))
    for s in (1, 2, 4, 8, 16):
        v = jnp.where((sub & s) != 0, jnp.roll(v, s, axis=1), v)

    mi = jnp.where(x == v, sub, _N)
    for s in (16, 8, 4, 2, 1):
        mi = jnp.minimum(mi, jnp.roll(mi, -s, axis=1))
    for s in (1, 2, 4, 8, 16):
        mi = jnp.where((sub & s) != 0, jnp.roll(mi, s, axis=1), mi)

    oh = (mi == sub).astype(jnp.int32)
    red = jnp.max(oh, axis=0, keepdims=True)
    red = jnp.maximum(red, jnp.roll(red, 64, axis=1))
    red = jnp.maximum(red, jnp.roll(red, 32, axis=1))

    qi = jax.lax.broadcasted_iota(jnp.int32, (TOP, L), 0)
    ci = jax.lax.broadcasted_iota(jnp.int32, (TOP, L), 1)
    a = jnp.where(ci // G == qi, jnp.broadcast_to(red.astype(jnp.float32), (TOP, L)), 0.0)
    ri = jax.lax.broadcasted_iota(jnp.int32, (L, L), 0)
    li = jax.lax.broadcasted_iota(jnp.int32, (L, L), 1)
    b = jnp.where(ri % G == li // _N, 1.0, 0.0).astype(jnp.float32)
    keep = (
        jax.lax.dot_general(a, b, (((1,), (0,)), ((), ())),
                            preferred_element_type=jnp.float32)
        > 0.5
    )

    fill_top = jnp.full((TOP, L), _FILL, jnp.float32)
    o_ref[0] = jnp.full((R, L), _FILL, jnp.float32)
    o_ref[0, :TOP, :] = jnp.where(keep, x[:TOP, :], fill_top)


@jax.jit
def kernel(X):
    B, M, N = X.shape
    R = M * N // 128
    Xv = X.reshape(B, R, 128)
    out = pl.pallas_call(
        _max_layer_kernel,
        grid=(B,),
        in_specs=[pl.BlockSpec((1, R, 128), lambda i: (i, 0, 0))],
        out_specs=pl.BlockSpec((1, R, 128), lambda i: (i, 0, 0)),
        out_shape=jax.ShapeDtypeStruct((B, R, 128), jnp.float32),
    )(Xv)
    return out.reshape(B, M, N)
